# baseline (speedup 1.0000x reference)
import functools

import jax
import jax.numpy as jnp
from jax.experimental import pallas as pl
from jax.experimental.pallas import tpu as pltpu
from jax.experimental.pallas import tpu_sc as plsc


def _router_body(hs_ref, gw_ref, logits_ref, l9_ref):
    l9 = jnp.dot(hs_ref[...], gw_ref[...], preferred_element_type=jnp.float32)
    E = gw_ref.shape[1] - 1
    logits_ref[...] = l9[:, :E]
    l9_ref[...] = l9


def _make_sc_gate(T, EP):
    E = EP - 1
    NW = 16
    ntok = T // NW
    mesh = plsc.VectorSubcoreMesh(core_axis_name="c", subcore_axis_name="s")

    @functools.partial(
        pl.kernel, mesh=mesh,
        compiler_params=pltpu.CompilerParams(needs_layout_passes=False),
        out_type=jax.ShapeDtypeStruct((T * EP,), jnp.float32),
        scratch_types=[pltpu.VMEM((ntok * EP,), jnp.float32),
                       pltpu.VMEM((ntok * EP,), jnp.float32)])
    def sc_gate(l9f_hbm, dwf_out, lv, dv):
        cid = jax.lax.axis_index("c")
        sid = jax.lax.axis_index("s")
        wid = sid * 2 + cid

        @pl.when(wid < NW)
        def _():
            _gate_work(l9f_hbm, dwf_out, lv, dv, wid)

    def _gate_work(l9f_hbm, dwf_out, lv, dv, wid):
        base = wid * ntok * EP
        pltpu.sync_copy(l9f_hbm.at[pl.ds(base, ntok * EP)], lv)
        lanes = jax.lax.broadcasted_iota(jnp.int32, (16,), 0)
        for c in range(ntok // 16):
            gx = [EP * (c * 16 + lanes) + e for e in range(EP)]
            v = [plsc.load_gather(lv, [gx[e]]) for e in range(EP)]
            mx = v[0]
            for e in range(1, E):
                mx = jnp.maximum(mx, v[e])
            ex = [jnp.exp(v[e] - mx) for e in range(E)]
            s = ex[0]
            for e in range(1, E):
                s = s + ex[e]
            rw = [ex[e] / s for e in range(E)]
            m1 = rw[0]
            for e in range(1, E):
                m1 = jnp.maximum(m1, rw[e])
            found = jnp.zeros((16,), jnp.bool_)
            sel0 = []
            for e in range(E):
                hit = jnp.logical_and(rw[e] == m1, jnp.logical_not(found))
                sel0.append(hit)
                found = jnp.logical_or(found, hit)
            r2 = [jnp.where(sel0[e], -1.0, rw[e]) for e in range(E)]
            m2 = r2[0]
            for e in range(1, E):
                m2 = jnp.maximum(m2, r2[e])
            found2 = jnp.zeros((16,), jnp.bool_)
            sel1 = []
            for e in range(E):
                hit = jnp.logical_and(r2[e] == m2, jnp.logical_not(found2))
                sel1.append(hit)
                found2 = jnp.logical_or(found2, hit)
            den = m1 + m2
            w0 = m1 / den
            w1 = m2 / den
            zero = jnp.zeros((16,), jnp.float32)
            for e in range(E):
                plsc.store_scatter(
                    dv, [gx[e]],
                    jnp.where(sel0[e], w0, jnp.where(sel1[e], w1, zero)))
            plsc.store_scatter(dv, [gx[E]], 1.0 / (1.0 + jnp.exp(-v[E])))
        pltpu.sync_copy(dv, dwf_out.at[pl.ds(base, ntok * EP)])

    return sc_gate


def _ffn_body(hs_ref, w0_ref, w1_ref, wo_ref, b0_ref, b1_ref, bo_ref, dw_ref,
              out_ref):
    e = pl.program_id(1)
    x = hs_ref[...].astype(jnp.bfloat16)
    h0 = jnp.dot(x, w0_ref[0], preferred_element_type=jnp.float32) + b0_ref[0]
    h1 = jnp.dot(x, w1_ref[0], preferred_element_type=jnp.float32) + b1_ref[0]
    inter = (h0 * jax.nn.sigmoid(h0) * h1).astype(jnp.bfloat16)
    out = jnp.dot(inter, wo_ref[0], preferred_element_type=jnp.float32) + bo_ref[0]
    lane = jax.lax.broadcasted_iota(jnp.int32, dw_ref.shape, 1)
    wcol = jnp.sum(jnp.where(lane == e, dw_ref[...], 0.0), axis=1, keepdims=True)
    contrib = out * wcol

    @pl.when(e == 0)
    def _():
        out_ref[...] = contrib

    @pl.when(e > 0)
    def _():
        out_ref[...] += contrib


def kernel(hidden_states, gate_w, W0, b0, W1, b1, Wo, bo, sW0, sb0, sW1, sb1,
           sWo, sbo, sg_w):
    b_, s_, h_ = hidden_states.shape
    T = b_ * s_
    E = gate_w.shape[1]
    I = W0.shape[2]
    EP = E + 1
    hs2 = hidden_states.reshape(T, h_)
    gwcat = jnp.concatenate([gate_w, sg_w], axis=1)

    logits, l9 = pl.pallas_call(
        _router_body,
        out_shape=[
            jax.ShapeDtypeStruct((T, E), jnp.float32),
            jax.ShapeDtypeStruct((T, EP), jnp.float32),
        ],
    )(hs2, gwcat)

    dw = _make_sc_gate(T, EP)(l9.reshape(T * EP)).reshape(T, EP)

    bf = jnp.bfloat16
    W0c = jnp.concatenate([W0, sW0[None]], axis=0).astype(bf)
    W1c = jnp.concatenate([W1, sW1[None]], axis=0).astype(bf)
    Woc = jnp.concatenate([Wo, sWo[None]], axis=0).astype(bf)
    b0c = jnp.concatenate([b0, sb0[None]], axis=0).reshape(EP, 1, I)
    b1c = jnp.concatenate([b1, sb1[None]], axis=0).reshape(EP, 1, I)
    boc = jnp.concatenate([bo, sbo[None]], axis=0).reshape(EP, 1, h_)

    Bm = 512
    grid = (T // Bm, EP)
    final = pl.pallas_call(
        _ffn_body,
        grid=grid,
        in_specs=[
            pl.BlockSpec((Bm, h_), lambda i, e: (i, 0)),
            pl.BlockSpec((1, h_, I), lambda i, e: (e, 0, 0)),
            pl.BlockSpec((1, h_, I), lambda i, e: (e, 0, 0)),
            pl.BlockSpec((1, I, h_), lambda i, e: (e, 0, 0)),
            pl.BlockSpec((1, 1, I), lambda i, e: (e, 0, 0)),
            pl.BlockSpec((1, 1, I), lambda i, e: (e, 0, 0)),
            pl.BlockSpec((1, 1, h_), lambda i, e: (e, 0, 0)),
            pl.BlockSpec((Bm, EP), lambda i, e: (i, 0)),
        ],
        out_specs=pl.BlockSpec((Bm, h_), lambda i, e: (i, 0)),
        out_shape=jax.ShapeDtypeStruct((T, h_), jnp.float32),
    )(hs2, W0c, W1c, Woc, b0c, b1c, boc, dw)

    return final.reshape(b_, s_, h_), logits


# speedup vs baseline: 1.0032x; 1.0032x over previous
import functools

import jax
import jax.numpy as jnp
from jax.experimental import pallas as pl
from jax.experimental.pallas import tpu as pltpu
from jax.experimental.pallas import tpu_sc as plsc


def _router_body(hs_ref, gw_ref, logits_ref, l9_ref):
    l9 = jnp.dot(hs_ref[...], gw_ref[...], preferred_element_type=jnp.float32)
    E = gw_ref.shape[1] - 1
    logits_ref[...] = l9[:, :E]
    l9_ref[...] = l9


def _make_sc_gate(T, EP):
    E = EP - 1
    NW = 32
    ntok = T // NW
    mesh = plsc.VectorSubcoreMesh(core_axis_name="c", subcore_axis_name="s")

    @functools.partial(
        pl.kernel, mesh=mesh,
        compiler_params=pltpu.CompilerParams(needs_layout_passes=False),
        out_type=jax.ShapeDtypeStruct((T * EP,), jnp.float32),
        scratch_types=[pltpu.VMEM((ntok * EP,), jnp.float32),
                       pltpu.VMEM((ntok * EP,), jnp.float32)])
    def sc_gate(l9f_hbm, dwf_out, lv, dv):
        cid = jax.lax.axis_index("c")
        sid = jax.lax.axis_index("s")
        wid = sid * 2 + cid

        @pl.when(wid < NW)
        def _():
            _gate_work(l9f_hbm, dwf_out, lv, dv, wid)

    def _gate_work(l9f_hbm, dwf_out, lv, dv, wid):
        base = wid * ntok * EP
        pltpu.sync_copy(l9f_hbm.at[pl.ds(base, ntok * EP)], lv)
        lanes = jax.lax.broadcasted_iota(jnp.int32, (16,), 0)
        for c in range(ntok // 16):
            gx = [EP * (c * 16 + lanes) + e for e in range(EP)]
            v = [plsc.load_gather(lv, [gx[e]]) for e in range(EP)]
            mx = v[0]
            for e in range(1, E):
                mx = jnp.maximum(mx, v[e])
            ex = [jnp.exp(v[e] - mx) for e in range(E)]
            s = ex[0]
            for e in range(1, E):
                s = s + ex[e]
            rw = [ex[e] / s for e in range(E)]
            m1 = rw[0]
            for e in range(1, E):
                m1 = jnp.maximum(m1, rw[e])
            found = jnp.zeros((16,), jnp.bool_)
            sel0 = []
            for e in range(E):
                hit = jnp.logical_and(rw[e] == m1, jnp.logical_not(found))
                sel0.append(hit)
                found = jnp.logical_or(found, hit)
            r2 = [jnp.where(sel0[e], -1.0, rw[e]) for e in range(E)]
            m2 = r2[0]
            for e in range(1, E):
                m2 = jnp.maximum(m2, r2[e])
            found2 = jnp.zeros((16,), jnp.bool_)
            sel1 = []
            for e in range(E):
                hit = jnp.logical_and(r2[e] == m2, jnp.logical_not(found2))
                sel1.append(hit)
                found2 = jnp.logical_or(found2, hit)
            den = m1 + m2
            w0 = m1 / den
            w1 = m2 / den
            zero = jnp.zeros((16,), jnp.float32)
            for e in range(E):
                plsc.store_scatter(
                    dv, [gx[e]],
                    jnp.where(sel0[e], w0, jnp.where(sel1[e], w1, zero)))
            plsc.store_scatter(dv, [gx[E]], 1.0 / (1.0 + jnp.exp(-v[E])))
        pltpu.sync_copy(dv, dwf_out.at[pl.ds(base, ntok * EP)])

    return sc_gate


def _ffn_body(hs_ref, w0_ref, w1_ref, wo_ref, b0_ref, b1_ref, bo_ref, dw_ref,
              out_ref):
    e = pl.program_id(1)
    x = hs_ref[...].astype(jnp.bfloat16)
    h0 = jnp.dot(x, w0_ref[0], preferred_element_type=jnp.float32) + b0_ref[0]
    h1 = jnp.dot(x, w1_ref[0], preferred_element_type=jnp.float32) + b1_ref[0]
    inter = (h0 * jax.nn.sigmoid(h0) * h1).astype(jnp.bfloat16)
    out = jnp.dot(inter, wo_ref[0], preferred_element_type=jnp.float32) + bo_ref[0]
    lane = jax.lax.broadcasted_iota(jnp.int32, dw_ref.shape, 1)
    wcol = jnp.sum(jnp.where(lane == e, dw_ref[...], 0.0), axis=1, keepdims=True)
    contrib = out * wcol

    @pl.when(e == 0)
    def _():
        out_ref[...] = contrib

    @pl.when(e > 0)
    def _():
        out_ref[...] += contrib


def kernel(hidden_states, gate_w, W0, b0, W1, b1, Wo, bo, sW0, sb0, sW1, sb1,
           sWo, sbo, sg_w):
    b_, s_, h_ = hidden_states.shape
    T = b_ * s_
    E = gate_w.shape[1]
    I = W0.shape[2]
    EP = E + 1
    hs2 = hidden_states.reshape(T, h_)
    gwcat = jnp.concatenate([gate_w, sg_w], axis=1)

    logits, l9 = pl.pallas_call(
        _router_body,
        out_shape=[
            jax.ShapeDtypeStruct((T, E), jnp.float32),
            jax.ShapeDtypeStruct((T, EP), jnp.float32),
        ],
    )(hs2, gwcat)

    dw = _make_sc_gate(T, EP)(l9.reshape(T * EP)).reshape(T, EP)

    bf = jnp.bfloat16
    W0c = jnp.concatenate([W0, sW0[None]], axis=0).astype(bf)
    W1c = jnp.concatenate([W1, sW1[None]], axis=0).astype(bf)
    Woc = jnp.concatenate([Wo, sWo[None]], axis=0).astype(bf)
    b0c = jnp.concatenate([b0, sb0[None]], axis=0).reshape(EP, 1, I)
    b1c = jnp.concatenate([b1, sb1[None]], axis=0).reshape(EP, 1, I)
    boc = jnp.concatenate([bo, sbo[None]], axis=0).reshape(EP, 1, h_)

    Bm = 512
    grid = (T // Bm, EP)
    final = pl.pallas_call(
        _ffn_body,
        grid=grid,
        in_specs=[
            pl.BlockSpec((Bm, h_), lambda i, e: (i, 0)),
            pl.BlockSpec((1, h_, I), lambda i, e: (e, 0, 0)),
            pl.BlockSpec((1, h_, I), lambda i, e: (e, 0, 0)),
            pl.BlockSpec((1, I, h_), lambda i, e: (e, 0, 0)),
            pl.BlockSpec((1, 1, I), lambda i, e: (e, 0, 0)),
            pl.BlockSpec((1, 1, I), lambda i, e: (e, 0, 0)),
            pl.BlockSpec((1, 1, h_), lambda i, e: (e, 0, 0)),
            pl.BlockSpec((Bm, EP), lambda i, e: (i, 0)),
        ],
        out_specs=pl.BlockSpec((Bm, h_), lambda i, e: (i, 0)),
        out_shape=jax.ShapeDtypeStruct((T, h_), jnp.float32),
    )(hs2, W0c, W1c, Woc, b0c, b1c, boc, dw)

    return final.reshape(b_, s_, h_), logits
